# baseline (speedup 1.0000x reference)
import functools

import jax
import jax.numpy as jnp
from jax import lax
from jax.experimental import pallas as pl
from jax.experimental.pallas import tpu as pltpu
from jax.experimental.pallas import tpu_sc as plsc

F = 26
V = 100000
D = 32
P = 128
B = 16384

NC = 2
NS = 16
NW = NC * NS
BPW = B // NW
NST = BPW // 128
NCHK = F * NST
GROUPS = F * V // 4
NSLOT = 7
LOOK = 3


def _sc_gather(tab_hbm, idx_hbm):
    mesh = plsc.VectorSubcoreMesh(core_axis_name="c", subcore_axis_name="s")

    @functools.partial(
        pl.kernel,
        mesh=mesh,
        out_type=jax.ShapeDtypeStruct((F, B, 128), jnp.float32),
        compiler_params=pltpu.CompilerParams(
            use_tc_tiling_on_sc=False, needs_layout_passes=False),
        scratch_types=[
            pltpu.VMEM((F, BPW), jnp.int32),
        ] + [pltpu.VMEM((128, 128), jnp.float32) for _ in range(NSLOT)]
          + [pltpu.SemaphoreType.DMA for _ in range(2 * NSLOT)],
    )
    def k(tab, idx, out, idx_v, *bufs_sems):
        bufs = bufs_sems[:NSLOT]
        gsems = bufs_sems[NSLOT:2 * NSLOT]
        osems = bufs_sems[2 * NSLOT:3 * NSLOT]
        wid = lax.axis_index("s") * NC + lax.axis_index("c")
        pltpu.sync_copy(idx.at[wid], idx_v)
        b0 = wid * BPW

        def src(c):
            f = c // NST
            s = c % NST
            return tab.at[idx_v.at[f, pl.ds(s * 128, 128)]]

        def dst(c):
            f = c // NST
            s = c % NST
            return out.at[f, pl.ds(b0 + s * 128, 128), pl.ds(0, 128)]

        def fire_g(c, slot):
            pltpu.async_copy(src(c), bufs[slot], gsems[slot])

        def wait_g(c, slot):
            pltpu.make_async_copy(src(c), bufs[slot], gsems[slot]).wait()

        def fire_o(c, slot):
            pltpu.async_copy(bufs[slot], dst(c), osems[slot])

        def wait_o(c, slot):
            pltpu.make_async_copy(bufs[slot], dst(c), osems[slot]).wait()

        def step(c, slot):
            @pl.when(jnp.logical_and(c >= NSLOT - LOOK, c + LOOK < NCHK))
            def _():
                wait_o(c + LOOK - NSLOT, (slot + LOOK) % NSLOT)

            @pl.when(c + LOOK < NCHK)
            def _():
                fire_g(c + LOOK, (slot + LOOK) % NSLOT)

            wait_g(c, slot)
            fire_o(c, slot)

        for c in range(LOOK):
            fire_g(c, c)

        def body(g, _):
            for p in range(NSLOT):
                step(g * NSLOT + p, p)
            return 0

        nfull = NCHK // NSLOT
        lax.fori_loop(0, nfull, body, 0)
        for c in range(nfull * NSLOT, NCHK):
            step(c, c % NSLOT)
        for c in range(NCHK - NSLOT, NCHK):
            wait_o(c, c % NSLOT)

    return k(tab_hbm, idx_hbm)


def _tc_body(x_ref, q_ref, g_ref, bt_ref, w1_ref, b1_ref, w2_ref, b2_ref,
             o_ref):
    x = x_ref[...]
    q = q_ref[...]
    lane = lax.broadcasted_iota(jnp.int32, x.shape, 2)
    m = (lane // D) == q[:, :, None]
    xm = jnp.where(m, x, 0.0)
    mu = jnp.sum(xm, axis=2, keepdims=True) * (1.0 / D)
    m2 = jnp.sum(xm * xm, axis=2, keepdims=True) * (1.0 / D)
    var = m2 - mu * mu
    h = (x - mu) * lax.rsqrt(var + 1e-5)
    h = jnp.where(m, h * g_ref[...][:, None, :] + bt_ref[...][:, None, :], 0.0)
    hb = h.astype(jnp.bfloat16)
    w1 = w1_ref[...]
    t = b1_ref[...].astype(jnp.float32)
    for f in range(F):
        t = t + jnp.dot(hb[f], w1[f], preferred_element_type=jnp.float32)
    u = 0.5 * t * (1.0 + lax.erf(t * 0.7071067811865476))
    o_ref[...] = jnp.dot(u.astype(jnp.bfloat16), w2_ref[...],
                         preferred_element_type=jnp.float32) + b2_ref[...]


def _tc_mlp(emb, qT, gamma, beta, w1r, b1, w2, b2, interpret=False):
    BT = 256
    grid = (B // BT,)
    return pl.pallas_call(
        _tc_body,
        grid=grid,
        in_specs=[
            pl.BlockSpec((F, BT, 128), lambda i: (0, i, 0)),
            pl.BlockSpec((F, BT), lambda i: (0, i)),
            pl.BlockSpec((F, 128), lambda i: (0, 0)),
            pl.BlockSpec((F, 128), lambda i: (0, 0)),
            pl.BlockSpec((F, 128, P), lambda i: (0, 0, 0)),
            pl.BlockSpec((1, P), lambda i: (0, 0)),
            pl.BlockSpec((P, P), lambda i: (0, 0)),
            pl.BlockSpec((1, P), lambda i: (0, 0)),
        ],
        out_specs=pl.BlockSpec((BT, P), lambda i: (i, 0)),
        out_shape=jax.ShapeDtypeStruct((B, P), jnp.float32),
        interpret=interpret,
    )(emb, qT, gamma, beta, w1r, b1, w2, b2)


def kernel(x_cat, tables, ln_gamma, ln_beta, W1, b1, W2, b2):
    offs = (jnp.arange(F, dtype=jnp.int32) * V)[None, :]
    gidx = ((x_cat + offs) // 4).reshape(NW, BPW, F).transpose(0, 2, 1)
    qT = (x_cat % 4).T
    tab4 = tables.reshape(GROUPS, 128)
    emb = _sc_gather(tab4, gidx)
    w1r = jnp.tile(W1.reshape(F, D, P), (1, 128 // D, 1)).astype(jnp.bfloat16)
    return _tc_mlp(
        emb, qT, jnp.tile(ln_gamma, (1, 128 // D)),
        jnp.tile(ln_beta, (1, 128 // D)),
        w1r, b1.reshape(1, P), W2.astype(jnp.bfloat16), b2.reshape(1, P),
    )
